# baseline (speedup 1.0000x reference)
import functools

import jax
import jax.numpy as jnp
from jax import lax
from jax.experimental import pallas as pl
from jax.experimental.pallas import tpu as pltpu
from jax.experimental.pallas import tpu_sc as plsc

EMBED = 32
BATCH = 16384
NROWS = 1000000

NC = 2
NS = 16
LANES = 16
EPT = BATCH // NS
GRP = EPT // LANES

W = 46080
NWIN = NROWS // W
NTILED = (NROWS // 128) * 128
TAIL = NTILED - NWIN * W
TDIR = NROWS - NTILED
RPW = 8 * W
HALF = 4 * RPW


def _gather_body(user, item, ut, it, tu, ti, part,
                 el_u, el_i, stage_u, stage_i, hit_lu, hit_li,
                 hit_bu, hit_bi, land, acc_v, shard, ssem, gsem):
    cid = lax.axis_index("c")
    sid = lax.axis_index("s")
    ebase = pl.multiple_of(sid * EPT, EPT)

    pltpu.sync_copy(user.at[pl.ds(ebase, EPT)], el_u)
    pltpu.sync_copy(item.at[pl.ds(ebase, EPT)], el_i)

    r0 = cid * 2

    def window_copies(w, db):
        hb = pl.multiple_of(db * HALF, 8)
        v0 = pl.multiple_of(w * W, 128)
        cps = []
        for t, tab in ((0, ut), (1, it)):
            for r2 in range(2):
                bb = (t * 2 + r2) * RPW
                for r8 in range(8):
                    cps.append(pltpu.make_async_copy(
                        tab.at[r0 + r2, r8, pl.ds(v0, W)],
                        shard.at[pl.ds(hb + bb + r8 * W, W)], ssem))
        return cps

    def start_window(w, db):
        for cp in window_copies(w, db):
            cp.start()

    def wait_window(w, db):
        for cp in window_copies(w, db):
            cp.wait()

    def gather_window(w, db, wl):
        def mkidx(g, carry):
            sl = pl.ds(pl.multiple_of(g * LANES, LANES), LANES)
            w0 = w * W
            for el, idx in ((el_u, hit_lu), (el_i, hit_li)):
                v = el[sl]
                loc = v - w0
                m = (loc >= 0) & (loc < wl)
                idx[sl] = jnp.where(m, loc, -1)
            return carry

        lax.fori_loop(0, GRP, mkidx, 0)

        hb = pl.multiple_of(db * HALF, 8)
        cps = []
        for t, idx, stage in ((0, hit_lu, stage_u), (1, hit_li, stage_i)):
            for r2 in range(2):
                bb = (t * 2 + r2) * RPW
                for r8 in range(8):
                    e = r2 * 8 + r8
                    src = shard.at[pl.ds(hb + bb + r8 * W, W)]
                    cps.append(pltpu.make_async_copy(
                        src.at[plsc.Indices(idx, ignored_value=-1)],
                        stage.at[pl.ds(e * EPT, EPT)], gsem))
        for cp in cps:
            cp.start()
        for cp in cps:
            cp.wait()

    def step(w, carry):
        @pl.when(sid == 0)
        def _():
            start_window(w, 0)
            wait_window(w, 0)
        plsc.subcore_barrier()
        gather_window(w, 0, W)
        plsc.subcore_barrier()
        return carry

    lax.fori_loop(0, NWIN, step, 0)

    @pl.when(sid == 0)
    def _():
        v0 = NWIN * W
        cps = []
        for t, tab in ((0, ut), (1, it)):
            for r2 in range(2):
                bb = (t * 2 + r2) * RPW
                for r8 in range(8):
                    cps.append(pltpu.make_async_copy(
                        tab.at[r0 + r2, r8, pl.ds(v0, TAIL)],
                        shard.at[pl.ds(bb + r8 * W, TAIL)], ssem))
        for cp in cps:
            cp.start()
        for cp in cps:
            cp.wait()
    plsc.subcore_barrier()
    gather_window(NWIN, 0, TAIL)

    def tailidx(g, carry):
        sl = pl.ds(pl.multiple_of(g * LANES, LANES), LANES)
        for el, idx in ((el_u, hit_lu), (el_i, hit_li)):
            loc = el[sl] - NTILED
            idx[sl] = jnp.where(loc >= 0, loc, -1)
        return carry

    lax.fori_loop(0, GRP, tailidx, 0)
    cps = []
    for tab, idx, stage in ((tu, hit_lu, stage_u), (ti, hit_li, stage_i)):
        for e in range(16):
            eg = cid * 16 + e
            src = tab.at[pl.ds(pl.multiple_of(eg * TDIR, 8), TDIR)]
            cps.append(pltpu.make_async_copy(
                src.at[plsc.Indices(idx, ignored_value=-1)],
                stage.at[pl.ds(e * EPT, EPT)], gsem))
    for cp in cps:
        cp.start()
    for cp in cps:
        cp.wait()

    def dot(g, carry):
        sl = pl.ds(pl.multiple_of(g * LANES, LANES), LANES)
        acc = jnp.zeros((LANES,), jnp.float32)
        for e in range(16):
            bsl = pl.ds(pl.multiple_of(e * EPT + g * LANES, LANES), LANES)
            acc = acc + stage_u[bsl] * stage_i[bsl]
        acc_v[sl] = acc
        return carry

    lax.fori_loop(0, GRP, dot, 0)
    pbase = pl.multiple_of(cid * BATCH + ebase, EPT)
    pltpu.sync_copy(acc_v, part.at[pl.ds(pbase, EPT)])


def _combine_body(part, out, p0, p1, out_v):
    wid = lax.axis_index("s") * NC + lax.axis_index("c")
    n = BATCH // (NC * NS)
    base = pl.multiple_of(wid * n, n)
    pltpu.sync_copy(part.at[pl.ds(base, n)], p0)
    pltpu.sync_copy(part.at[pl.ds(BATCH + base, n)], p1)

    def chunk(g, carry):
        sl = pl.ds(pl.multiple_of(g * LANES, LANES), LANES)
        s = p0[sl] + p1[sl]
        out_v[sl] = 1.0 / (1.0 + jnp.exp(-s))
        return carry

    lax.fori_loop(0, n // LANES, chunk, 0)
    pltpu.sync_copy(out_v, out.at[pl.ds(base, n)])


@jax.jit
def _pmf_forward(user, item, user_table, item_table):
    mesh = plsc.VectorSubcoreMesh(core_axis_name="c", subcore_axis_name="s")
    gather = functools.partial(
        pl.kernel,
        out_type=jax.ShapeDtypeStruct((NC * BATCH,), jnp.float32),
        mesh=mesh,
        scratch_types=[
            pltpu.VMEM((EPT,), jnp.int32),
            pltpu.VMEM((EPT,), jnp.int32),
            pltpu.VMEM((16 * EPT,), jnp.float32),
            pltpu.VMEM((16 * EPT,), jnp.float32),
            pltpu.VMEM((EPT,), jnp.int32),
            pltpu.VMEM((EPT,), jnp.int32),
            pltpu.VMEM((EPT,), jnp.int32),
            pltpu.VMEM((EPT,), jnp.int32),
            pltpu.VMEM((16 * 128,), jnp.float32),
            pltpu.VMEM((EPT,), jnp.float32),
            pltpu.VMEM_SHARED((HALF,), jnp.float32),
            pltpu.SemaphoreType.DMA,
            pltpu.SemaphoreType.DMA,
        ],
        compiler_params=pltpu.CompilerParams(use_tc_tiling_on_sc=True),
    )(_gather_body)
    combine = functools.partial(
        pl.kernel,
        out_type=jax.ShapeDtypeStruct((BATCH,), jnp.float32),
        mesh=mesh,
        scratch_types=[
            pltpu.VMEM((BATCH // (NC * NS),), jnp.float32),
            pltpu.VMEM((BATCH // (NC * NS),), jnp.float32),
            pltpu.VMEM((BATCH // (NC * NS),), jnp.float32),
        ],
        compiler_params=pltpu.CompilerParams(use_tc_tiling_on_sc=True),
    )(_combine_body)
    ut = user_table.T.reshape(4, 8, NROWS)
    it = item_table.T.reshape(4, 8, NROWS)
    tu = user_table[NTILED:].T.reshape(-1)
    ti = item_table[NTILED:].T.reshape(-1)
    part = gather(user, item, ut, it, tu, ti)
    return combine(part)


def kernel(user, item, user_table, item_table):
    return _pmf_forward(user, item, user_table, item_table)


# speedup vs baseline: 1.4626x; 1.4626x over previous
import functools

import jax
import jax.numpy as jnp
from jax import lax
from jax.experimental import pallas as pl
from jax.experimental.pallas import tpu as pltpu
from jax.experimental.pallas import tpu_sc as plsc

EMBED = 32
BATCH = 16384
NROWS = 1000000

NC = 2
NS = 16
LANES = 16
EPT = BATCH // NS
GRP = EPT // LANES

W = 46080
NWIN = NROWS // W
NTILED = (NROWS // 128) * 128
TAIL = NTILED - NWIN * W
TDIR = NROWS - NTILED
RPW = 8 * W
HALF = 4 * RPW


def _gather_body(user, item, ut, it, tu, ti, part,
                 el_u, el_i, stage_u, stage_i, hit_lu, hit_li,
                 hit_bu, hit_bi, land, acc_v, shard, ssem, gsem):
    cid = lax.axis_index("c")
    sid = lax.axis_index("s")
    ebase = pl.multiple_of(sid * EPT, EPT)

    pltpu.sync_copy(user.at[pl.ds(ebase, EPT)], el_u)
    pltpu.sync_copy(item.at[pl.ds(ebase, EPT)], el_i)

    r0 = cid * 2

    def tab_copies(t, w):
        tab = (ut, it)[t]
        v0 = pl.multiple_of(w * W, 128)
        cps = []
        for r2 in range(2):
            bb = (t * 2 + r2) * RPW
            for r8 in range(8):
                cps.append(pltpu.make_async_copy(
                    tab.at[r0 + r2, r8, pl.ds(v0, W)],
                    shard.at[pl.ds(bb + r8 * W, W)], ssem))
        return cps

    def start_tab(t, w):
        for cp in tab_copies(t, w):
            cp.start()

    def wait_tab(t, w):
        for cp in tab_copies(t, w):
            cp.wait()

    def gather_one(w, wl, t, el, idx, stage):
        def mkidx(g, carry):
            sl = pl.ds(pl.multiple_of(g * LANES, LANES), LANES)
            w0 = w * W
            v = el[sl]
            loc = v - w0
            m = (loc >= 0) & (loc < wl)
            idx[sl] = jnp.where(m, loc, -1)
            return carry

        lax.fori_loop(0, GRP, mkidx, 0)

        cps = []
        for r2 in range(2):
            bb = (t * 2 + r2) * RPW
            for r8 in range(8):
                e = r2 * 8 + r8
                src = shard.at[pl.ds(bb + r8 * W, W)]
                cps.append(pltpu.make_async_copy(
                    src.at[plsc.Indices(idx, ignored_value=-1)],
                    stage.at[pl.ds(e * EPT, EPT)], gsem))
        for cp in cps:
            cp.start()
        for cp in cps:
            cp.wait()

    def gather_window(w, db, wl):
        gather_one(w, wl, 0, el_u, hit_lu, stage_u)
        gather_one(w, wl, 1, el_i, hit_li, stage_i)

    @pl.when(sid == 0)
    def _():
        start_tab(0, 0)
        wait_tab(0, 0)
        start_tab(1, 0)
    plsc.subcore_barrier()

    def step(w, carry):
        gather_one(w, W, 0, el_u, hit_lu, stage_u)
        plsc.subcore_barrier()

        @pl.when(sid == 0)
        def _():
            wait_tab(1, w)

            @pl.when(w + 1 < NWIN)
            def _():
                start_tab(0, w + 1)
        plsc.subcore_barrier()
        gather_one(w, W, 1, el_i, hit_li, stage_i)
        plsc.subcore_barrier()

        @pl.when((sid == 0) & (w + 1 < NWIN))
        def _():
            wait_tab(0, w + 1)
            start_tab(1, w + 1)
        plsc.subcore_barrier()
        return carry

    lax.fori_loop(0, NWIN, step, 0)

    @pl.when(sid == 0)
    def _():
        v0 = NWIN * W
        cps = []
        for t, tab in ((0, ut), (1, it)):
            for r2 in range(2):
                bb = (t * 2 + r2) * RPW
                for r8 in range(8):
                    cps.append(pltpu.make_async_copy(
                        tab.at[r0 + r2, r8, pl.ds(v0, TAIL)],
                        shard.at[pl.ds(bb + r8 * W, TAIL)], ssem))
        for cp in cps:
            cp.start()
        for cp in cps:
            cp.wait()
    plsc.subcore_barrier()
    gather_window(NWIN, 0, TAIL)

    def tailidx(g, carry):
        sl = pl.ds(pl.multiple_of(g * LANES, LANES), LANES)
        for el, idx in ((el_u, hit_lu), (el_i, hit_li)):
            loc = el[sl] - NTILED
            idx[sl] = jnp.where(loc >= 0, loc, -1)
        return carry

    lax.fori_loop(0, GRP, tailidx, 0)
    cps = []
    for tab, idx, stage in ((tu, hit_lu, stage_u), (ti, hit_li, stage_i)):
        for e in range(16):
            eg = cid * 16 + e
            src = tab.at[pl.ds(pl.multiple_of(eg * TDIR, 8), TDIR)]
            cps.append(pltpu.make_async_copy(
                src.at[plsc.Indices(idx, ignored_value=-1)],
                stage.at[pl.ds(e * EPT, EPT)], gsem))
    for cp in cps:
        cp.start()
    for cp in cps:
        cp.wait()

    def dot(g, carry):
        sl = pl.ds(pl.multiple_of(g * LANES, LANES), LANES)
        acc = jnp.zeros((LANES,), jnp.float32)
        for e in range(16):
            bsl = pl.ds(pl.multiple_of(e * EPT + g * LANES, LANES), LANES)
            acc = acc + stage_u[bsl] * stage_i[bsl]
        acc_v[sl] = acc
        return carry

    lax.fori_loop(0, GRP, dot, 0)
    pbase = pl.multiple_of(cid * BATCH + ebase, EPT)
    pltpu.sync_copy(acc_v, part.at[pl.ds(pbase, EPT)])


def _combine_body(part, out, p0, p1, out_v):
    wid = lax.axis_index("s") * NC + lax.axis_index("c")
    n = BATCH // (NC * NS)
    base = pl.multiple_of(wid * n, n)
    pltpu.sync_copy(part.at[pl.ds(base, n)], p0)
    pltpu.sync_copy(part.at[pl.ds(BATCH + base, n)], p1)

    def chunk(g, carry):
        sl = pl.ds(pl.multiple_of(g * LANES, LANES), LANES)
        s = p0[sl] + p1[sl]
        out_v[sl] = 1.0 / (1.0 + jnp.exp(-s))
        return carry

    lax.fori_loop(0, n // LANES, chunk, 0)
    pltpu.sync_copy(out_v, out.at[pl.ds(base, n)])


@jax.jit
def _pmf_forward(user, item, user_table, item_table):
    mesh = plsc.VectorSubcoreMesh(core_axis_name="c", subcore_axis_name="s")
    gather = functools.partial(
        pl.kernel,
        out_type=jax.ShapeDtypeStruct((NC * BATCH,), jnp.float32),
        mesh=mesh,
        scratch_types=[
            pltpu.VMEM((EPT,), jnp.int32),
            pltpu.VMEM((EPT,), jnp.int32),
            pltpu.VMEM((16 * EPT,), jnp.float32),
            pltpu.VMEM((16 * EPT,), jnp.float32),
            pltpu.VMEM((EPT,), jnp.int32),
            pltpu.VMEM((EPT,), jnp.int32),
            pltpu.VMEM((EPT,), jnp.int32),
            pltpu.VMEM((EPT,), jnp.int32),
            pltpu.VMEM((16 * 128,), jnp.float32),
            pltpu.VMEM((EPT,), jnp.float32),
            pltpu.VMEM_SHARED((HALF,), jnp.float32),
            pltpu.SemaphoreType.DMA,
            pltpu.SemaphoreType.DMA,
        ],
        compiler_params=pltpu.CompilerParams(use_tc_tiling_on_sc=True),
    )(_gather_body)
    combine = functools.partial(
        pl.kernel,
        out_type=jax.ShapeDtypeStruct((BATCH,), jnp.float32),
        mesh=mesh,
        scratch_types=[
            pltpu.VMEM((BATCH // (NC * NS),), jnp.float32),
            pltpu.VMEM((BATCH // (NC * NS),), jnp.float32),
            pltpu.VMEM((BATCH // (NC * NS),), jnp.float32),
        ],
        compiler_params=pltpu.CompilerParams(use_tc_tiling_on_sc=True),
    )(_combine_body)
    ut = user_table.T.reshape(4, 8, NROWS)
    it = item_table.T.reshape(4, 8, NROWS)
    tu = user_table[NTILED:].T.reshape(-1)
    ti = item_table[NTILED:].T.reshape(-1)
    part = gather(user, item, ut, it, tu, ti)
    return combine(part)


def kernel(user, item, user_table, item_table):
    return _pmf_forward(user, item, user_table, item_table)
